# Initial kernel scaffold; baseline (speedup 1.0000x reference)
import jax
import jax.numpy as jnp
from jax.experimental import pallas as pl


def kernel(image, efficiency_map, xlors, ylors, zlors):
    raise NotImplementedError("write your pallas kernel here")


# speedup vs baseline: 89.9339x; 89.9339x over previous
import functools

import jax
import jax.numpy as jnp
from jax import lax
from jax.experimental import pallas as pl
from jax.experimental.pallas import tpu as pltpu
from jax.experimental.pallas import tpu_sc as plsc

GRID_N = 128
NVOX = GRID_N * GRID_N * GRID_N
NS = 64
EPS = 1e-8
NLORS = 300000

NC = 2
NSUB = 16
NW = NC * NSUB
HALF = NVOX // NC

NPAD = 307200
PT = NPAD // NW
GROUPS_A = PT // 16
PC = NPAD // NSUB
LCH = 2400
GROUPS_B = LCH // 16
NCH_B = PC // LCH
GSAMP = 16 * NS
TILE_VOX = HALF // NSUB


def _seg_body(lt_ref, seg_ref):
  dx = lt_ref[3, :] - lt_ref[0, :]
  dy = lt_ref[4, :] - lt_ref[1, :]
  dz = lt_ref[5, :] - lt_ref[2, :]
  seg_ref[0, :] = jnp.sqrt(dx * dx + dy * dy + dz * dz) / jnp.float32(NS)


def _seg_tc(lorsT):
  return pl.pallas_call(
      _seg_body,
      out_shape=jax.ShapeDtypeStruct((1, NPAD), jnp.float32),
  )(lorsT)


def _combine_body(img_ref, eff_ref, bp_ref, out_ref):
  out_ref[...] = img_ref[...] / (eff_ref[...] + EPS) * bp_ref[...]


def _combine_tc(img2d, eff2d, bp2d):
  nrow = img2d.shape[0]
  blk = nrow // 8
  spec = pl.BlockSpec((blk, 128), lambda i: (i, 0))
  return pl.pallas_call(
      _combine_body,
      grid=(8,),
      in_specs=[spec, spec, spec],
      out_specs=spec,
      out_shape=jax.ShapeDtypeStruct((nrow, 128), jnp.float32),
  )(img2d, eff2d, bp2d)


def _sample_flat(lt_vs, tb_v, g, k):
  sl = pl.ds(g * 16, 16)
  p0x = lt_vs[0][sl] * 128.0
  p0y = lt_vs[1][sl] * 128.0
  p0z = lt_vs[2][sl] * 128.0
  dx = (lt_vs[3][sl] - lt_vs[0][sl]) * 128.0
  dy = (lt_vs[4][sl] - lt_vs[1][sl]) * 128.0
  dz = (lt_vs[5][sl] - lt_vs[2][sl]) * 128.0
  tv = tb_v[pl.ds(k * 16, 16)]
  fx = p0x + tv * dx
  fy = p0y + tv * dy
  fz = p0z + tv * dz
  mn = jnp.minimum(fx, jnp.minimum(fy, fz))
  mx = jnp.maximum(fx, jnp.maximum(fy, fz))
  inb = (mn >= 0.0) & (mx < 128.0)
  ix = fx.astype(jnp.int32)
  iy = fy.astype(jnp.int32)
  iz = fz.astype(jnp.int32)
  flat = (ix * GRID_N + iy) * GRID_N + iz
  return flat, inb


_MESH = plsc.VectorSubcoreMesh(core_axis_name="c", subcore_axis_name="s")

_LOR_VMEM_A = [pltpu.VMEM((PT,), jnp.float32)] * 6
_LOR_VMEM_B = [pltpu.VMEM((LCH,), jnp.float32)] * 6


@functools.partial(
    pl.kernel,
    out_type=jax.ShapeDtypeStruct((NPAD,), jnp.float32),
    mesh=_MESH,
    scratch_types=[
        *_LOR_VMEM_A,
        pltpu.VMEM((PT,), jnp.float32),
        pltpu.VMEM((PT,), jnp.float32),
        pltpu.VMEM((NS * 16,), jnp.float32),
        pltpu.VMEM((GSAMP,), jnp.int32),
        pltpu.VMEM((GSAMP,), jnp.float32),
        pltpu.VMEM((GSAMP,), jnp.float32),
    ],
)
def _project_sc(l0, l1, l2, l3, l4, l5, seg_hbm, tb_hbm, img_hbm, w_hbm,
                v0, v1, v2, v3, v4, v5, seg_v, w_v, tb_v, idx_v, msk_v, val_v):
  c = lax.axis_index("c")
  s = lax.axis_index("s")
  wid = c * NSUB + s
  base = wid * PT
  lt_vs = (v0, v1, v2, v3, v4, v5)
  for lr, vr in zip((l0, l1, l2, l3, l4, l5), lt_vs):
    pltpu.sync_copy(lr.at[pl.ds(base, PT)], vr)
  pltpu.sync_copy(seg_hbm.at[pl.ds(base, PT)], seg_v)
  pltpu.sync_copy(tb_hbm, tb_v)

  def group_body(g, carry):
    def idx_body(k, carry):
      flat, inb = _sample_flat(lt_vs, tb_v, g, k)
      gidx = jnp.clip(flat, 0, NVOX - 1)
      ks = pl.ds(k * 16, 16)
      idx_v[ks] = gidx
      msk_v[ks] = jnp.where(inb, 1.0, 0.0).astype(jnp.float32)
      return carry

    lax.fori_loop(0, NS, idx_body, 0)
    pltpu.sync_copy(img_hbm.at[idx_v], val_v)

    def acc_body(k, acc):
      ks = pl.ds(k * 16, 16)
      return acc + val_v[ks] * msk_v[ks]

    acc = lax.fori_loop(0, NS, acc_body, jnp.zeros((16,), jnp.float32))
    sl = pl.ds(g * 16, 16)
    seg16 = seg_v[sl]
    proj = acc * seg16
    w_v[sl] = seg16 / (proj + EPS)
    return carry

  lax.fori_loop(0, GROUPS_A, group_body, 0)
  pltpu.sync_copy(w_v, w_hbm.at[pl.ds(base, PT)])


@functools.partial(
    pl.kernel,
    out_type=jax.ShapeDtypeStruct((NVOX,), jnp.float32),
    mesh=_MESH,
    scratch_types=[
        pltpu.VMEM_SHARED((HALF,), jnp.float32),
        *_LOR_VMEM_B,
        pltpu.VMEM((LCH,), jnp.float32),
        pltpu.VMEM((NS * 16,), jnp.float32),
        pltpu.VMEM((GSAMP,), jnp.int32),
        pltpu.VMEM((GSAMP,), jnp.float32),
        pltpu.VMEM((8192,), jnp.float32),
    ],
)
def _backproject_sc(l0, l1, l2, l3, l4, l5, w_hbm, tb_hbm, bp_hbm,
                    acc_sh, v0, v1, v2, v3, v4, v5, w_v, tb_v, idx_v, val_v,
                    z_v):
  c = lax.axis_index("c")
  s = lax.axis_index("s")
  vbase = c * HALF
  lt_vs = (v0, v1, v2, v3, v4, v5)

  def zfill(i, carry):
    z_v[pl.ds(i * 16, 16)] = jnp.zeros((16,), jnp.float32)
    return carry

  lax.fori_loop(0, 8192 // 16, zfill, 0)
  for j in range(TILE_VOX // 8192):
    pltpu.sync_copy(z_v, acc_sh.at[pl.ds(s * TILE_VOX + j * 8192, 8192)])
  pltpu.sync_copy(tb_hbm, tb_v)
  plsc.subcore_barrier()

  def chunk_body(ch, carry):
    lbase = s * PC + ch * LCH
    for lr, vr in zip((l0, l1, l2, l3, l4, l5), lt_vs):
      pltpu.sync_copy(lr.at[pl.ds(lbase, LCH)], vr)
    pltpu.sync_copy(w_hbm.at[pl.ds(lbase, LCH)], w_v)

    def group_body(g, carry):
      w16 = w_v[pl.ds(g * 16, 16)]

      def idx_body(k, carry):
        flat, inb = _sample_flat(lt_vs, tb_v, g, k)
        local = flat - vbase
        ok = inb & (local.astype(jnp.uint32) < jnp.uint32(HALF))
        ks = pl.ds(k * 16, 16)
        idx_v[ks] = jnp.where(ok, local, -1)
        val_v[ks] = w16
        return carry

      lax.fori_loop(0, NS, idx_body, 0)
      pltpu.sync_copy(
          val_v, acc_sh.at[plsc.Indices(idx_v, ignored_value=-1)], add=True)
      return carry

    lax.fori_loop(0, GROUPS_B, group_body, 0)
    return carry

  lax.fori_loop(0, NCH_B, chunk_body, 0)
  plsc.subcore_barrier()
  pltpu.sync_copy(acc_sh.at[pl.ds(s * TILE_VOX, TILE_VOX)],
                  bp_hbm.at[pl.ds(vbase + s * TILE_VOX, TILE_VOX)])


def kernel(image, efficiency_map, xlors, ylors, zlors):
  lors = jnp.concatenate([xlors, ylors, zlors], axis=0)
  npad = NPAD - NLORS
  h = (jnp.arange(npad, dtype=jnp.float32) % 1024.0) / 1024.0
  padlors = jnp.stack([h, h, h, h, h, h], axis=1)
  lorsT = jnp.concatenate([lors, padlors], axis=0).T
  lcomp = [lorsT[r] for r in range(6)]
  t = jnp.linspace(0.0, 1.0, NS, dtype=jnp.float32)
  tb = jnp.broadcast_to(t[:, None], (NS, 16)).reshape(NS * 16)

  seg = _seg_tc(lorsT).reshape(NPAD)
  w = _project_sc(*lcomp, seg, tb, image.reshape(NVOX))
  bp = _backproject_sc(*lcomp, w, tb)
  out = _combine_tc(image.reshape(NVOX // 128, 128),
                    efficiency_map.reshape(NVOX // 128, 128),
                    bp.reshape(NVOX // 128, 128))
  return out.reshape(GRID_N, GRID_N, GRID_N)


# speedup vs baseline: 132.3007x; 1.4711x over previous
import functools

import jax
import jax.numpy as jnp
from jax import lax
from jax.experimental import pallas as pl
from jax.experimental.pallas import tpu as pltpu
from jax.experimental.pallas import tpu_sc as plsc

GRID_N = 128
NVOX = GRID_N * GRID_N * GRID_N
NS = 64
EPS = 1e-8
NLORS = 300000

NC = 2
NSUB = 16
NW = NC * NSUB
HALF = NVOX // NC

NPAD = 307200
PT = NPAD // NW
GROUPS_A = PT // 16
PC = NPAD // NSUB
SCH = 320
GROUPS_B = SCH // 16
NCH_B = PC // SCH
GSAMP = 16 * NS
FLCH = SCH * NS
TILE_VOX = HALF // NSUB
NSAMP = NPAD * NS


def _seg_body(lt_ref, seg_ref):
  dx = lt_ref[3, :] - lt_ref[0, :]
  dy = lt_ref[4, :] - lt_ref[1, :]
  dz = lt_ref[5, :] - lt_ref[2, :]
  seg_ref[0, :] = jnp.sqrt(dx * dx + dy * dy + dz * dz) / jnp.float32(NS)


def _seg_tc(lorsT):
  return pl.pallas_call(
      _seg_body,
      out_shape=jax.ShapeDtypeStruct((1, NPAD), jnp.float32),
  )(lorsT)


def _combine_body(img_ref, eff_ref, bp_ref, out_ref):
  out_ref[...] = img_ref[...] / (eff_ref[...] + EPS) * bp_ref[...]


def _combine_tc(img2d, eff2d, bp2d):
  nrow = img2d.shape[0]
  blk = nrow // 8
  spec = pl.BlockSpec((blk, 128), lambda i: (i, 0))
  return pl.pallas_call(
      _combine_body,
      grid=(8,),
      in_specs=[spec, spec, spec],
      out_specs=spec,
      out_shape=jax.ShapeDtypeStruct((nrow, 128), jnp.float32),
  )(img2d, eff2d, bp2d)


def _sample_flat(lt_vs, tb_v, g, k):
  sl = pl.ds(g * 16, 16)
  p0x = lt_vs[0][sl] * 128.0
  p0y = lt_vs[1][sl] * 128.0
  p0z = lt_vs[2][sl] * 128.0
  dx = (lt_vs[3][sl] - lt_vs[0][sl]) * 128.0
  dy = (lt_vs[4][sl] - lt_vs[1][sl]) * 128.0
  dz = (lt_vs[5][sl] - lt_vs[2][sl]) * 128.0
  tv = tb_v[pl.ds(k * 16, 16)]
  fx = p0x + tv * dx
  fy = p0y + tv * dy
  fz = p0z + tv * dz
  mn = jnp.minimum(fx, jnp.minimum(fy, fz))
  mx = jnp.maximum(fx, jnp.maximum(fy, fz))
  inb = (mn >= 0.0) & (mx < 128.0)
  ix = fx.astype(jnp.int32)
  iy = fy.astype(jnp.int32)
  iz = fz.astype(jnp.int32)
  flat = (ix * GRID_N + iy) * GRID_N + iz
  return flat, inb


_MESH = plsc.VectorSubcoreMesh(core_axis_name="c", subcore_axis_name="s")

_LOR_VMEM_A = [pltpu.VMEM((PT,), jnp.float32)] * 6


@functools.partial(
    pl.kernel,
    out_type=(jax.ShapeDtypeStruct((NPAD,), jnp.float32),
              jax.ShapeDtypeStruct((NSAMP,), jnp.int32)),
    mesh=_MESH,
    scratch_types=[
        *_LOR_VMEM_A,
        pltpu.VMEM((PT,), jnp.float32),
        pltpu.VMEM((PT,), jnp.float32),
        pltpu.VMEM((NS * 16,), jnp.float32),
        pltpu.VMEM((GSAMP,), jnp.int32),
        pltpu.VMEM((GSAMP,), jnp.int32),
        pltpu.VMEM((GSAMP,), jnp.int32),
        pltpu.VMEM((GSAMP,), jnp.int32),
        pltpu.VMEM((GSAMP,), jnp.float32),
        pltpu.VMEM((GSAMP,), jnp.float32),
        pltpu.VMEM((GSAMP,), jnp.float32),
        pltpu.VMEM((GSAMP,), jnp.float32),
        pltpu.SemaphoreType.DMA,
        pltpu.SemaphoreType.DMA,
        pltpu.SemaphoreType.DMA,
        pltpu.SemaphoreType.DMA,
    ],
)
def _project_sc(l0, l1, l2, l3, l4, l5, seg_hbm, tb_hbm, img_hbm,
                w_hbm, fl_hbm,
                v0, v1, v2, v3, v4, v5, seg_v, w_v, tb_v,
                idx0, idx1, fl0, fl1, msk0, msk1, val0, val1,
                sem0, sem1, semf0, semf1):
  c = lax.axis_index("c")
  s = lax.axis_index("s")
  wid = c * NSUB + s
  base = wid * PT
  fbase = base * NS
  lt_vs = (v0, v1, v2, v3, v4, v5)
  for lr, vr in zip((l0, l1, l2, l3, l4, l5), lt_vs):
    pltpu.sync_copy(lr.at[pl.ds(base, PT)], vr)
  pltpu.sync_copy(seg_hbm.at[pl.ds(base, PT)], seg_v)
  pltpu.sync_copy(tb_hbm, tb_v)

  idx_b = (idx0, idx1)
  fl_b = (fl0, fl1)
  msk_b = (msk0, msk1)
  val_b = (val0, val1)
  sem_b = (sem0, sem1)
  semf_b = (semf0, semf1)

  def compute_idx(g, b):
    @pl.loop(0, NS, unroll=4)
    def idx_body(k):
      flat, inb = _sample_flat(lt_vs, tb_v, g, k)
      gidx = jnp.clip(flat, 0, NVOX - 1)
      ks = pl.ds(k * 16, 16)
      idx_b[b][ks] = gidx
      fl_b[b][ks] = jnp.where(inb, flat, -1)
      msk_b[b][ks] = jnp.where(inb, 1.0, 0.0).astype(jnp.float32)

  def fire(g, b):
    pltpu.async_copy(img_hbm.at[idx_b[b]], val_b[b], sem_b[b])
    pltpu.async_copy(fl_b[b], fl_hbm.at[pl.ds(fbase + g * GSAMP, GSAMP)],
                     semf_b[b])

  def finish(g, b):
    pltpu.make_async_copy(img_hbm.at[idx_b[b]], val_b[b], sem_b[b]).wait()
    pltpu.make_async_copy(
        fl_b[b], fl_hbm.at[pl.ds(fbase + g * GSAMP, GSAMP)], semf_b[b]).wait()

    def acc_body(k, acc):
      ks = pl.ds(k * 16, 16)
      return acc + val_b[b][ks] * msk_b[b][ks]

    acc = lax.fori_loop(0, NS, acc_body, jnp.zeros((16,), jnp.float32),
                        unroll=4)
    sl = pl.ds(g * 16, 16)
    seg16 = seg_v[sl]
    proj = acc * seg16
    w_v[sl] = seg16 / (proj + EPS)

  compute_idx(0, 0)
  fire(0, 0)

  @pl.loop(0, GROUPS_A, step=2)
  def group_body(g):
    compute_idx(g + 1, 1)
    fire(g + 1, 1)
    finish(g, 0)

    @pl.when(g + 2 < GROUPS_A)
    def _():
      compute_idx(g + 2, 0)
      fire(g + 2, 0)

    finish(g + 1, 1)

  pltpu.sync_copy(w_v, w_hbm.at[pl.ds(base, PT)])


@functools.partial(
    pl.kernel,
    out_type=jax.ShapeDtypeStruct((NVOX,), jnp.float32),
    mesh=_MESH,
    scratch_types=[
        pltpu.VMEM_SHARED((HALF,), jnp.float32),
        pltpu.VMEM((FLCH,), jnp.int32),
        pltpu.VMEM((FLCH,), jnp.int32),
        pltpu.VMEM((SCH,), jnp.float32),
        pltpu.VMEM((SCH,), jnp.float32),
        pltpu.VMEM((GSAMP,), jnp.int32),
        pltpu.VMEM((GSAMP,), jnp.int32),
        pltpu.VMEM((GSAMP,), jnp.float32),
        pltpu.VMEM((GSAMP,), jnp.float32),
        pltpu.VMEM((8192,), jnp.float32),
        pltpu.SemaphoreType.DMA,
        pltpu.SemaphoreType.DMA,
        pltpu.SemaphoreType.DMA,
        pltpu.SemaphoreType.DMA,
    ],
)
def _backproject_sc(w_hbm, fl_hbm, bp_hbm,
                    acc_sh, flc0, flc1, wc0, wc1,
                    idx0, idx1, val0, val1, z_v,
                    sem0, sem1, semc0, semc1):
  c = lax.axis_index("c")
  s = lax.axis_index("s")
  vbase = c * HALF
  flc_b = (flc0, flc1)
  wc_b = (wc0, wc1)
  idx_b = (idx0, idx1)
  val_b = (val0, val1)
  sem_b = (sem0, sem1)
  semc_b = (semc0, semc1)

  def zfill(i, carry):
    z_v[pl.ds(i * 16, 16)] = jnp.zeros((16,), jnp.float32)
    return carry

  lax.fori_loop(0, 8192 // 16, zfill, 0)
  for j in range(TILE_VOX // 8192):
    pltpu.sync_copy(z_v, acc_sh.at[pl.ds(s * TILE_VOX + j * 8192, 8192)])
  plsc.subcore_barrier()

  def fetch_chunk(ch, b):
    lbase = s * PC + ch * SCH
    pltpu.async_copy(fl_hbm.at[pl.ds(lbase * NS, FLCH)], flc_b[b], semc_b[b])
    pltpu.async_copy(w_hbm.at[pl.ds(lbase, SCH)], wc_b[b], semc_b[b])

  def wait_chunk(ch, b):
    lbase = s * PC + ch * SCH
    pltpu.make_async_copy(
        fl_hbm.at[pl.ds(lbase * NS, FLCH)], flc_b[b], semc_b[b]).wait()
    pltpu.make_async_copy(
        w_hbm.at[pl.ds(lbase, SCH)], wc_b[b], semc_b[b]).wait()

  def compute_idx(cb, g, b):
    w16 = wc_b[cb][pl.ds(g * 16, 16)]

    @pl.loop(0, NS, unroll=8)
    def idx_body(k):
      fl16 = flc_b[cb][pl.ds(g * GSAMP + k * 16, 16)]
      local = fl16 - vbase
      ok = local.astype(jnp.uint32) < jnp.uint32(HALF)
      ks = pl.ds(k * 16, 16)
      idx_b[b][ks] = jnp.where(ok, local, -1)
      val_b[b][ks] = w16

  def fire(b):
    pltpu.async_copy(
        val_b[b], acc_sh.at[plsc.Indices(idx_b[b], ignored_value=-1)],
        sem_b[b], add=True)

  def wait(b):
    pltpu.make_async_copy(
        val_b[b], acc_sh.at[plsc.Indices(idx_b[b], ignored_value=-1)],
        sem_b[b]).wait()

  def process(cb):
    def do(g, b):
      compute_idx(cb, g, b)
      fire(b)

    do(0, 0)

    @pl.loop(0, GROUPS_B, step=2)
    def group_body(g):
      do(g + 1, 1)
      wait(0)

      @pl.when(g + 2 < GROUPS_B)
      def _():
        do(g + 2, 0)

      wait(1)

  fetch_chunk(0, 0)
  fetch_chunk(1, 1)

  @pl.loop(0, NCH_B, step=2)
  def chunk_body(ch):
    wait_chunk(ch, 0)
    process(0)

    @pl.when(ch + 2 < NCH_B)
    def _():
      fetch_chunk(ch + 2, 0)

    wait_chunk(ch + 1, 1)
    process(1)

    @pl.when(ch + 3 < NCH_B)
    def _():
      fetch_chunk(ch + 3, 1)

  plsc.subcore_barrier()
  pltpu.sync_copy(acc_sh.at[pl.ds(s * TILE_VOX, TILE_VOX)],
                  bp_hbm.at[pl.ds(vbase + s * TILE_VOX, TILE_VOX)])


def kernel(image, efficiency_map, xlors, ylors, zlors):
  lors = jnp.concatenate([xlors, ylors, zlors], axis=0)
  npad = NPAD - NLORS
  h = jnp.arange(npad, dtype=jnp.float32) / jnp.float32(npad)
  h2 = (jnp.arange(npad, dtype=jnp.float32) + 1.0) / jnp.float32(npad)
  padlors = jnp.stack([h, h, -2.0 - h, h2, h2, -2.0 - h2], axis=1)
  lorsT = jnp.concatenate([lors, padlors], axis=0).T
  lcomp = [lorsT[r] for r in range(6)]
  t = jnp.linspace(0.0, 1.0, NS, dtype=jnp.float32)
  tb = jnp.broadcast_to(t[:, None], (NS, 16)).reshape(NS * 16)

  seg = _seg_tc(lorsT).reshape(NPAD)
  w, fl = _project_sc(*lcomp, seg, tb, image.reshape(NVOX))
  bp = _backproject_sc(w, fl)
  out = _combine_tc(image.reshape(NVOX // 128, 128),
                    efficiency_map.reshape(NVOX // 128, 128),
                    bp.reshape(NVOX // 128, 128))
  return out.reshape(GRID_N, GRID_N, GRID_N)
